# baseline (speedup 1.0000x reference)
import jax
import jax.numpy as jnp
from jax.experimental import pallas as pl
from jax.experimental.pallas import tpu as pltpu

_SEQ_BLOCK = 1024


def _emb_kernel(inp_ref, wk_hbm, wks_hbm, bc_ref, bss_ref, out_ref,
                wk_scr, wks_scr, sem):
    i = pl.program_id(0)

    @pl.when(i == 0)
    def _load_tables():
        cp1 = pltpu.make_async_copy(wk_hbm, wk_scr, sem.at[0])
        cp2 = pltpu.make_async_copy(wks_hbm, wks_scr, sem.at[1])
        cp1.start()
        cp2.start()
        cp1.wait()
        cp2.wait()

    tab = wk_scr[...] * bc_ref[0] + wks_scr[...] * bss_ref[0]
    m = (inp_ref[...] != 0).astype(tab.dtype)
    out_ref[...] = tab[None, :, :] * m[:, :, None]


def kernel(input_tensor, weights):
    batch, seq_len = input_tensor.shape
    dim = weights.shape[1]
    s_blk = _SEQ_BLOCK if seq_len % _SEQ_BLOCK == 0 else seq_len
    n_blk = seq_len // s_blk

    wk = jax.lax.slice(weights, (1, 0), (1 + s_blk, dim))
    wks = wk.reshape(s_blk, dim // 2, 2)[:, :, ::-1].reshape(s_blk, dim)

    base = weights[jnp.arange(n_blk) * s_blk]
    base = base.at[0].set(jnp.tile(jnp.array([0.0, 1.0], weights.dtype),
                                   dim // 2))
    pairs = base.reshape(n_blk, dim // 2, 2)
    sin_b = pairs[:, :, 0:1]
    cos_b = pairs[:, :, 1:2]
    bc = jnp.broadcast_to(cos_b, (n_blk, dim // 2, 2)).reshape(n_blk, dim)
    sign = jnp.tile(jnp.array([1.0, -1.0], weights.dtype), dim // 2)
    bss = jnp.broadcast_to(sin_b, (n_blk, dim // 2, 2)).reshape(n_blk, dim)
    bss = bss * sign
    bc = bc.reshape(n_blk, 1, dim)
    bss = bss.reshape(n_blk, 1, dim)

    out = pl.pallas_call(
        _emb_kernel,
        grid=(n_blk,),
        in_specs=[
            pl.BlockSpec((batch, s_blk), lambda i: (0, i)),
            pl.BlockSpec(memory_space=pltpu.MemorySpace.HBM),
            pl.BlockSpec(memory_space=pltpu.MemorySpace.HBM),
            pl.BlockSpec((1, 1, dim), lambda i: (i, 0, 0)),
            pl.BlockSpec((1, 1, dim), lambda i: (i, 0, 0)),
        ],
        out_specs=pl.BlockSpec((batch, s_blk, dim), lambda i: (0, i, 0)),
        out_shape=jax.ShapeDtypeStruct((batch, seq_len, dim), weights.dtype),
        scratch_shapes=[
            pltpu.VMEM((s_blk, dim), weights.dtype),
            pltpu.VMEM((s_blk, dim), weights.dtype),
            pltpu.SemaphoreType.DMA((2,)),
        ],
        compiler_params=pltpu.CompilerParams(
            dimension_semantics=("arbitrary",),
        ),
    )(input_tensor, wk, wks, bc, bss)
    return out


# speedup vs baseline: 1.6611x; 1.6611x over previous
import functools
import jax
import jax.numpy as jnp
from jax.experimental import pallas as pl
from jax.experimental.pallas import tpu as pltpu

_SEQ_BLOCK = 1024


def _emb_kernel(inp_ref, w_ref, nxt_ref, out_ref, *, s_blk):
    w_blk = w_ref[...]
    rolled = pltpu.roll(w_blk, s_blk - 1, 0)
    row_id = jax.lax.broadcasted_iota(jnp.int32, w_blk.shape, 0)
    w = jnp.where(row_id == s_blk - 1, nxt_ref[0], rolled)
    m = (inp_ref[...] != 0).astype(w.dtype)
    out_ref[...] = w[None, :, :] * m[:, :, None]


def kernel(input_tensor, weights):
    batch, seq_len = input_tensor.shape
    dim = weights.shape[1]
    s_blk = _SEQ_BLOCK if seq_len % _SEQ_BLOCK == 0 else seq_len
    n_blk = seq_len // s_blk

    nxt = weights[(jnp.arange(n_blk) + 1) * s_blk].reshape(n_blk, 1, dim)

    out = pl.pallas_call(
        functools.partial(_emb_kernel, s_blk=s_blk),
        grid=(n_blk,),
        in_specs=[
            pl.BlockSpec((batch, s_blk), lambda i: (0, i)),
            pl.BlockSpec((s_blk, dim), lambda i: (i, 0)),
            pl.BlockSpec((1, 1, dim), lambda i: (i, 0, 0)),
        ],
        out_specs=pl.BlockSpec((batch, s_blk, dim), lambda i: (0, i, 0)),
        out_shape=jax.ShapeDtypeStruct((batch, seq_len, dim), weights.dtype),
        compiler_params=pltpu.CompilerParams(
            dimension_semantics=("arbitrary",),
        ),
    )(input_tensor, weights, nxt)
    return out
